# Initial kernel scaffold; baseline (speedup 1.0000x reference)
import jax
import jax.numpy as jnp
from jax.experimental import pallas as pl


def kernel(x, edge_index, batch, Wq, bq, Wk, bk, Wv, bv, Ws, bs, Wout, bout):
    raise NotImplementedError("write your pallas kernel here")


# speedup vs baseline: 6.5280x; 6.5280x over previous
import functools

import jax
import jax.numpy as jnp
from jax import lax
from jax.experimental import pallas as pl
from jax.experimental.pallas import tpu as pltpu
from jax.experimental.pallas import tpu_sc as plsc

N = 10000
E = 320000
C = 128
HEADS = 8
HC = 16
D = 128
OUT = 128
L = 8

NW = 32
NODES_PER_W = 320
NPAD = NW * NODES_PER_W
OFFS_PAD = NPAD + 16
NEG = -1e30


def _proj_body(hp_ref, agg_ref, sp_ref, w_ref, b_ref, h_ref, q_ref, kv_ref,
               spo_ref):
    h = jnp.maximum(agg_ref[...] + sp_ref[...], 0.0) + hp_ref[...]
    h_ref[...] = h
    z = jnp.dot(h, w_ref[...], preferred_element_type=jnp.float32) + b_ref[...][0]
    q_ref[...] = z[:, :128]
    kv_ref[...] = z[:, 128:384]
    spo_ref[...] = z[:, 384:]


def _tc_proj(h_prev, agg, sp, wcat, bcat):
    blk = 512
    grid = (NPAD // blk,)
    return pl.pallas_call(
        _proj_body,
        grid=grid,
        in_specs=[
            pl.BlockSpec((blk, 128), lambda i: (i, 0)),
            pl.BlockSpec((blk, 128), lambda i: (i, 0)),
            pl.BlockSpec((blk, 128), lambda i: (i, 0)),
            pl.BlockSpec((128, 512), lambda i: (0, 0)),
            pl.BlockSpec((8, 512), lambda i: (0, 0)),
        ],
        out_specs=[
            pl.BlockSpec((blk, 128), lambda i: (i, 0)),
            pl.BlockSpec((blk, 128), lambda i: (i, 0)),
            pl.BlockSpec((blk, 256), lambda i: (i, 0)),
            pl.BlockSpec((blk, 128), lambda i: (i, 0)),
        ],
        out_shape=[
            jax.ShapeDtypeStruct((NPAD, 128), jnp.float32),
            jax.ShapeDtypeStruct((NPAD, 128), jnp.float32),
            jax.ShapeDtypeStruct((NPAD, 256), jnp.float32),
            jax.ShapeDtypeStruct((NPAD, 128), jnp.float32),
        ],
    )(h_prev, agg, sp, wcat, bcat)


def _final_body(hp_ref, agg_ref, sp_ref, w_ref, b_ref, o_ref):
    h = jnp.maximum(agg_ref[...] + sp_ref[...], 0.0) + hp_ref[...]
    o_ref[...] = jnp.dot(h, w_ref[...], preferred_element_type=jnp.float32) \
        + b_ref[...][0]


def _tc_final(h_prev, agg, sp, wout, bout):
    blk = 512
    grid = (NPAD // blk,)
    return pl.pallas_call(
        _final_body,
        grid=grid,
        in_specs=[
            pl.BlockSpec((blk, 128), lambda i: (i, 0)),
            pl.BlockSpec((blk, 128), lambda i: (i, 0)),
            pl.BlockSpec((blk, 128), lambda i: (i, 0)),
            pl.BlockSpec((128, OUT), lambda i: (0, 0)),
            pl.BlockSpec((8, OUT), lambda i: (0, 0)),
        ],
        out_specs=pl.BlockSpec((blk, OUT), lambda i: (i, 0)),
        out_shape=jax.ShapeDtypeStruct((NPAD, OUT), jnp.float32),
    )(h_prev, agg, sp, wout, bout)


def _sc_body(q_hbm, kv_hbm, srcs_hbm, offs_hbm, out_hbm,
             offs_v, q_v, out_v, idx_v, kv_v, st_m, st_s, st_a, sem):
    wid = lax.axis_index("s") * 2 + lax.axis_index("c")
    nstart = wid * NODES_PER_W

    pltpu.sync_copy(offs_hbm.at[pl.ds(nstart, NODES_PER_W + 16)], offs_v)
    pltpu.sync_copy(q_hbm.at[pl.ds(nstart * 128, NODES_PER_W * 128)], q_v)

    lanes = lax.iota(jnp.int32, 16)

    def node_body(i, _):
        oo = offs_v[pl.ds(i, 16)]
        s = oo[0]
        e = oo[1]
        base = (s // 16) * 16
        ntiles = jnp.where(e > s, (e - base + 15) // 16, 0)

        def init_body(h, _):
            h16 = h * 16
            st_m[pl.ds(h16, 16)] = jnp.full((16,), NEG, jnp.float32)
            st_s[pl.ds(h16, 16)] = jnp.zeros((16,), jnp.float32)
            st_a[pl.ds(h16, 16)] = jnp.zeros((16,), jnp.float32)
            return 0

        lax.fori_loop(0, HEADS, init_body, 0)

        def tile_alpha(h16, valid):
            qh = q_v[pl.ds(i * 128 + h16, 16)]
            ps = []
            for c in range(HC):
                col = plsc.load_gather(
                    kv_v, [lanes, jnp.full((16,), h16 + c, jnp.int32)])
                ps.append(qh[c] * col)
            while len(ps) > 1:
                ps = [ps[k] + ps[k + 1] for k in range(0, len(ps), 2)]
            return jnp.where(valid, ps[0], NEG)

        def fetch_tile(t):
            g0 = pl.multiple_of(base + t * 16, 16)
            pltpu.sync_copy(srcs_hbm.at[pl.ds(g0, 16)], idx_v)
            pltpu.async_copy(kv_hbm.at[idx_v], kv_v, sem).wait()
            gvec = g0 + lanes
            return (gvec >= s) & (gvec < e)

        def max_body(t, _):
            valid = fetch_tile(t)

            def head_body(h, _):
                h16 = h * 16
                alpha = tile_alpha(h16, valid)
                st_m[pl.ds(h16, 16)] = jnp.maximum(st_m[pl.ds(h16, 16)], alpha)
                return 0

            lax.fori_loop(0, HEADS, head_body, 0)
            return 0

        lax.fori_loop(0, ntiles, max_body, 0)

        def red_body(h, _):
            h16 = h * 16
            st_m[pl.ds(h16, 16)] = jnp.full(
                (16,), jnp.max(st_m[pl.ds(h16, 16)]), jnp.float32)
            return 0

        lax.fori_loop(0, HEADS, red_body, 0)

        def den_body(t, _):
            valid = fetch_tile(t)

            def head_body(h, _):
                h16 = h * 16
                alpha = tile_alpha(h16, valid)
                w = jnp.where(valid, jnp.exp(alpha - st_m[pl.ds(h16, 16)]), 0.0)
                d = st_s[pl.ds(h16, 16)][0]
                for j in range(16):
                    d = d + w[j]
                st_s[pl.ds(h16, 16)] = jnp.full((16,), d, jnp.float32)
                return 0

            lax.fori_loop(0, HEADS, head_body, 0)
            return 0

        lax.fori_loop(0, ntiles, den_body, 0)

        def acc_body(t, _):
            valid = fetch_tile(t)

            def head_body(h, _):
                h16 = h * 16
                alpha = tile_alpha(h16, valid)
                w = jnp.where(valid, jnp.exp(alpha - st_m[pl.ds(h16, 16)]), 0.0)
                attn = w / (st_s[pl.ds(h16, 16)] + 1e-16)
                acc = st_a[pl.ds(h16, 16)]
                for j in range(16):
                    acc = acc + attn[j] * kv_v[j, pl.ds(128 + h16, 16)]
                st_a[pl.ds(h16, 16)] = acc
                return 0

            lax.fori_loop(0, HEADS, head_body, 0)
            return 0

        lax.fori_loop(0, ntiles, acc_body, 0)

        def fin_body(h, _):
            h16 = h * 16
            out_v[pl.ds(i * 128 + h16, 16)] = st_a[pl.ds(h16, 16)]
            return 0

        lax.fori_loop(0, HEADS, fin_body, 0)
        return 0

    lax.fori_loop(0, NODES_PER_W, node_body, 0)
    pltpu.sync_copy(out_v, out_hbm.at[pl.ds(nstart * 128, NODES_PER_W * 128)])


@functools.partial(
    pl.kernel,
    out_type=jax.ShapeDtypeStruct((NPAD * 128,), jnp.float32),
    mesh=plsc.VectorSubcoreMesh(core_axis_name="c", subcore_axis_name="s"),
    compiler_params=pltpu.CompilerParams(needs_layout_passes=False),
    scratch_types=[
        pltpu.VMEM((NODES_PER_W + 16,), jnp.int32),
        pltpu.VMEM((NODES_PER_W * 128,), jnp.float32),
        pltpu.VMEM((NODES_PER_W * 128,), jnp.float32),
        pltpu.VMEM((16,), jnp.int32),
        pltpu.VMEM((16, 256), jnp.float32),
        pltpu.VMEM((128,), jnp.float32),
        pltpu.VMEM((128,), jnp.float32),
        pltpu.VMEM((128,), jnp.float32),
        pltpu.SemaphoreType.DMA,
    ],
)
def _sc_agg(q_hbm, kv_hbm, srcs_hbm, offs_hbm, out_hbm,
            offs_v, q_v, out_v, idx_v, kv_v, st_m, st_s, st_a, sem):
    _sc_body(q_hbm, kv_hbm, srcs_hbm, offs_hbm, out_hbm,
             offs_v, q_v, out_v, idx_v, kv_v, st_m, st_s, st_a, sem)


def kernel(x, edge_index, batch, Wq, bq, Wk, bk, Wv, bv, Ws, bs, Wout, bout):
    src = edge_index[0]
    dst = edge_index[1]
    perm = jnp.argsort(dst)
    ssrc = src[perm].astype(jnp.int32)
    sdst = dst[perm]
    offs = jnp.searchsorted(
        sdst, jnp.arange(OFFS_PAD, dtype=jnp.int32), side="left"
    ).astype(jnp.int32)

    x_pad = jnp.pad(x, ((0, NPAD - N), (0, 0)))
    wcat = jnp.concatenate([Wq * 0.25, Wk, Wv, Ws], axis=2)
    bcat = jnp.broadcast_to(
        jnp.concatenate([bq * 0.25, bk, bv, bs], axis=1)[:, None, :],
        (L, 8, 512))

    def layer(carry, lw):
        h, agg, sp = carry
        wc, bc = lw
        h, q, kv, sp = _tc_proj(h, agg, sp, wc, bc)
        agg = _sc_agg(q.reshape(-1), kv, ssrc, offs).reshape(NPAD, 128)
        return (h, agg, sp), 0.0

    init = (x_pad, jnp.zeros((NPAD, 128), jnp.float32),
            jnp.zeros((NPAD, 128), jnp.float32))
    (h, agg, sp), _ = lax.scan(layer, init, (wcat, bcat))
    out = _tc_final(h, agg, sp, Wout, jnp.broadcast_to(bout, (8, OUT)))
    return out[:N]


# speedup vs baseline: 9.7687x; 1.4964x over previous
import functools

import jax
import jax.numpy as jnp
from jax import lax
from jax.experimental import pallas as pl
from jax.experimental.pallas import tpu as pltpu
from jax.experimental.pallas import tpu_sc as plsc

N = 10000
E = 320000
C = 128
HEADS = 8
HC = 16
D = 128
OUT = 128
L = 8

NW = 32
NODES_PER_W = 320
NPAD = NW * NODES_PER_W
OFFS_PAD = NPAD + 16
NEG = -1e30
CACHE_T = 6


def _proj_body(hp_ref, agg_ref, sp_ref, w_ref, b_ref, h_ref, q_ref, kv_ref,
               spo_ref):
    h = jnp.maximum(agg_ref[...] + sp_ref[...], 0.0) + hp_ref[...]
    h_ref[...] = h
    z = jnp.dot(h, w_ref[...], preferred_element_type=jnp.float32) + b_ref[...][0]
    q_ref[...] = z[:, :128]
    kv_ref[...] = z[:, 128:384]
    spo_ref[...] = z[:, 384:]


def _tc_proj(h_prev, agg, sp, wcat, bcat):
    blk = 512
    grid = (NPAD // blk,)
    return pl.pallas_call(
        _proj_body,
        grid=grid,
        in_specs=[
            pl.BlockSpec((blk, 128), lambda i: (i, 0)),
            pl.BlockSpec((blk, 128), lambda i: (i, 0)),
            pl.BlockSpec((blk, 128), lambda i: (i, 0)),
            pl.BlockSpec((128, 512), lambda i: (0, 0)),
            pl.BlockSpec((8, 512), lambda i: (0, 0)),
        ],
        out_specs=[
            pl.BlockSpec((blk, 128), lambda i: (i, 0)),
            pl.BlockSpec((blk, 128), lambda i: (i, 0)),
            pl.BlockSpec((blk, 256), lambda i: (i, 0)),
            pl.BlockSpec((blk, 128), lambda i: (i, 0)),
        ],
        out_shape=[
            jax.ShapeDtypeStruct((NPAD, 128), jnp.float32),
            jax.ShapeDtypeStruct((NPAD, 128), jnp.float32),
            jax.ShapeDtypeStruct((NPAD, 256), jnp.float32),
            jax.ShapeDtypeStruct((NPAD, 128), jnp.float32),
        ],
    )(h_prev, agg, sp, wcat, bcat)


def _final_body(hp_ref, agg_ref, sp_ref, w_ref, b_ref, o_ref):
    h = jnp.maximum(agg_ref[...] + sp_ref[...], 0.0) + hp_ref[...]
    o_ref[...] = jnp.dot(h, w_ref[...], preferred_element_type=jnp.float32) \
        + b_ref[...][0]


def _tc_final(h_prev, agg, sp, wout, bout):
    blk = 512
    grid = (NPAD // blk,)
    return pl.pallas_call(
        _final_body,
        grid=grid,
        in_specs=[
            pl.BlockSpec((blk, 128), lambda i: (i, 0)),
            pl.BlockSpec((blk, 128), lambda i: (i, 0)),
            pl.BlockSpec((blk, 128), lambda i: (i, 0)),
            pl.BlockSpec((128, OUT), lambda i: (0, 0)),
            pl.BlockSpec((8, OUT), lambda i: (0, 0)),
        ],
        out_specs=pl.BlockSpec((blk, OUT), lambda i: (i, 0)),
        out_shape=jax.ShapeDtypeStruct((NPAD, OUT), jnp.float32),
    )(h_prev, agg, sp, wout, bout)


def _sc_body(q_hbm, kv_hbm, srcs_hbm, offs_hbm, out_hbm,
             offs_v, q_v, out_v, idx_v, kv_v, st_m, st_s, st_a, sem):
    wid = lax.axis_index("s") * 2 + lax.axis_index("c")
    nstart = wid * NODES_PER_W

    pltpu.sync_copy(offs_hbm.at[pl.ds(nstart, NODES_PER_W + 16)], offs_v)
    pltpu.sync_copy(q_hbm.at[pl.ds(nstart * 128, NODES_PER_W * 128)], q_v)

    lanes = lax.iota(jnp.int32, 16)

    def node_body(i, _):
        oo = offs_v[pl.ds(i, 16)]
        s = oo[0]
        e = oo[1]
        base = (s // 16) * 16
        ntiles = jnp.where(e > s, (e - base + 15) // 16, 0)

        def init_body(h, _):
            h16 = h * 16
            st_m[pl.ds(h16, 16)] = jnp.full((16,), NEG, jnp.float32)
            st_s[pl.ds(h16, 16)] = jnp.zeros((16,), jnp.float32)
            st_a[pl.ds(h16, 16)] = jnp.zeros((16,), jnp.float32)
            return 0

        lax.fori_loop(0, HEADS, init_body, 0)

        refetch = ntiles > CACHE_T

        def tile_alpha(slot, h16, valid):
            qh = q_v[pl.ds(i * 128 + h16, 16)]
            rows = slot * 16 + lanes
            ps = []
            for c in range(HC):
                col = plsc.load_gather(
                    kv_v, [rows, jnp.full((16,), h16 + c, jnp.int32)])
                ps.append(qh[c] * col)
            while len(ps) > 1:
                ps = [ps[k] + ps[k + 1] for k in range(0, len(ps), 2)]
            return jnp.where(valid, ps[0], NEG)

        def fetch_tile(t, slot):
            g0 = pl.multiple_of(base + t * 16, 16)
            pltpu.sync_copy(srcs_hbm.at[pl.ds(g0, 16)], idx_v)
            pltpu.async_copy(
                kv_hbm.at[idx_v], kv_v.at[pl.ds(slot * 16, 16)], sem).wait()

        def tile_valid(t):
            gvec = base + t * 16 + lanes
            return (gvec >= s) & (gvec < e)

        def max_body(t, _):
            slot = lax.rem(t, CACHE_T)
            fetch_tile(t, slot)
            valid = tile_valid(t)

            def head_body(h, _):
                h16 = h * 16
                alpha = tile_alpha(slot, h16, valid)
                st_m[pl.ds(h16, 16)] = jnp.maximum(st_m[pl.ds(h16, 16)], alpha)
                return 0

            lax.fori_loop(0, HEADS, head_body, 0)
            return 0

        lax.fori_loop(0, ntiles, max_body, 0)

        def red_body(h, _):
            h16 = h * 16
            st_m[pl.ds(h16, 16)] = jnp.full(
                (16,), jnp.max(st_m[pl.ds(h16, 16)]), jnp.float32)
            return 0

        lax.fori_loop(0, HEADS, red_body, 0)

        def den_body(t, _):
            slot = lax.rem(t, CACHE_T)
            pl.when(refetch)(lambda: fetch_tile(t, slot))
            valid = tile_valid(t)

            def head_body(h, _):
                h16 = h * 16
                alpha = tile_alpha(slot, h16, valid)
                w = jnp.where(valid, jnp.exp(alpha - st_m[pl.ds(h16, 16)]), 0.0)
                d = st_s[pl.ds(h16, 16)][0]
                for j in range(16):
                    d = d + w[j]
                st_s[pl.ds(h16, 16)] = jnp.full((16,), d, jnp.float32)
                return 0

            lax.fori_loop(0, HEADS, head_body, 0)
            return 0

        lax.fori_loop(0, ntiles, den_body, 0)

        def acc_body(t, _):
            slot = lax.rem(t, CACHE_T)
            pl.when(refetch)(lambda: fetch_tile(t, slot))
            valid = tile_valid(t)

            def head_body(h, _):
                h16 = h * 16
                alpha = tile_alpha(slot, h16, valid)
                w = jnp.where(valid, jnp.exp(alpha - st_m[pl.ds(h16, 16)]), 0.0)
                attn = w / (st_s[pl.ds(h16, 16)] + 1e-16)
                acc = st_a[pl.ds(h16, 16)]
                vcols = 128 + h16 + lanes
                for j in range(16):
                    vrow = plsc.load_gather(
                        kv_v, [slot * 16 + j + jnp.zeros((16,), jnp.int32),
                               vcols])
                    acc = acc + attn[j] * vrow
                st_a[pl.ds(h16, 16)] = acc
                return 0

            lax.fori_loop(0, HEADS, head_body, 0)
            return 0

        lax.fori_loop(0, ntiles, acc_body, 0)

        def fin_body(h, _):
            h16 = h * 16
            out_v[pl.ds(i * 128 + h16, 16)] = st_a[pl.ds(h16, 16)]
            return 0

        lax.fori_loop(0, HEADS, fin_body, 0)
        return 0

    lax.fori_loop(0, NODES_PER_W, node_body, 0)
    pltpu.sync_copy(out_v, out_hbm.at[pl.ds(nstart * 128, NODES_PER_W * 128)])


@functools.partial(
    pl.kernel,
    out_type=jax.ShapeDtypeStruct((NPAD * 128,), jnp.float32),
    mesh=plsc.VectorSubcoreMesh(core_axis_name="c", subcore_axis_name="s"),
    compiler_params=pltpu.CompilerParams(needs_layout_passes=False),
    scratch_types=[
        pltpu.VMEM((NODES_PER_W + 16,), jnp.int32),
        pltpu.VMEM((NODES_PER_W * 128,), jnp.float32),
        pltpu.VMEM((NODES_PER_W * 128,), jnp.float32),
        pltpu.VMEM((16,), jnp.int32),
        pltpu.VMEM((CACHE_T * 16, 256), jnp.float32),
        pltpu.VMEM((128,), jnp.float32),
        pltpu.VMEM((128,), jnp.float32),
        pltpu.VMEM((128,), jnp.float32),
        pltpu.SemaphoreType.DMA,
    ],
)
def _sc_agg(q_hbm, kv_hbm, srcs_hbm, offs_hbm, out_hbm,
            offs_v, q_v, out_v, idx_v, kv_v, st_m, st_s, st_a, sem):
    _sc_body(q_hbm, kv_hbm, srcs_hbm, offs_hbm, out_hbm,
             offs_v, q_v, out_v, idx_v, kv_v, st_m, st_s, st_a, sem)


def kernel(x, edge_index, batch, Wq, bq, Wk, bk, Wv, bv, Ws, bs, Wout, bout):
    src = edge_index[0]
    dst = edge_index[1]
    perm = jnp.argsort(dst)
    ssrc = src[perm].astype(jnp.int32)
    sdst = dst[perm]
    offs = jnp.searchsorted(
        sdst, jnp.arange(OFFS_PAD, dtype=jnp.int32), side="left"
    ).astype(jnp.int32)

    x_pad = jnp.pad(x, ((0, NPAD - N), (0, 0)))
    wcat = jnp.concatenate([Wq * 0.25, Wk, Wv, Ws], axis=2)
    bcat = jnp.broadcast_to(
        jnp.concatenate([bq * 0.25, bk, bv, bs], axis=1)[:, None, :],
        (L, 8, 512))

    def layer(carry, lw):
        h, agg, sp = carry
        wc, bc = lw
        h, q, kv, sp = _tc_proj(h, agg, sp, wc, bc)
        agg = _sc_agg(q.reshape(-1), kv, ssrc, offs).reshape(NPAD, 128)
        return (h, agg, sp), 0.0

    init = (x_pad, jnp.zeros((NPAD, 128), jnp.float32),
            jnp.zeros((NPAD, 128), jnp.float32))
    (h, agg, sp), _ = lax.scan(layer, init, (wcat, bcat))
    out = _tc_final(h, agg, sp, Wout, jnp.broadcast_to(bout, (8, OUT)))
    return out[:N]


# speedup vs baseline: 10.2748x; 1.0518x over previous
import functools

import jax
import jax.numpy as jnp
from jax import lax
from jax.experimental import pallas as pl
from jax.experimental.pallas import tpu as pltpu
from jax.experimental.pallas import tpu_sc as plsc

N = 10000
E = 320000
C = 128
HEADS = 8
HC = 16
D = 128
OUT = 128
L = 8

NW = 32
NODES_PER_W = 320
NPAD = NW * NODES_PER_W
OFFS_PAD = NPAD + 16
NEG = -1e30
CACHE_T = 6


def _proj_body(hp_ref, agg_ref, sp_ref, w_ref, b_ref, h_ref, q_ref, kv_ref,
               spo_ref):
    h = jnp.maximum(agg_ref[...] + sp_ref[...], 0.0) + hp_ref[...]
    h_ref[...] = h
    z = jnp.dot(h, w_ref[...], preferred_element_type=jnp.float32) + b_ref[...][0]
    q_ref[...] = z[:, :128]
    kv_ref[...] = z[:, 128:384]
    spo_ref[...] = z[:, 384:]


def _tc_proj(h_prev, agg, sp, wcat, bcat):
    blk = 512
    grid = (NPAD // blk,)
    return pl.pallas_call(
        _proj_body,
        grid=grid,
        in_specs=[
            pl.BlockSpec((blk, 128), lambda i: (i, 0)),
            pl.BlockSpec((blk, 128), lambda i: (i, 0)),
            pl.BlockSpec((blk, 128), lambda i: (i, 0)),
            pl.BlockSpec((128, 512), lambda i: (0, 0)),
            pl.BlockSpec((8, 512), lambda i: (0, 0)),
        ],
        out_specs=[
            pl.BlockSpec((blk, 128), lambda i: (i, 0)),
            pl.BlockSpec((blk, 128), lambda i: (i, 0)),
            pl.BlockSpec((blk, 256), lambda i: (i, 0)),
            pl.BlockSpec((blk, 128), lambda i: (i, 0)),
        ],
        out_shape=[
            jax.ShapeDtypeStruct((NPAD, 128), jnp.float32),
            jax.ShapeDtypeStruct((NPAD, 128), jnp.float32),
            jax.ShapeDtypeStruct((NPAD, 256), jnp.float32),
            jax.ShapeDtypeStruct((NPAD, 128), jnp.float32),
        ],
    )(h_prev, agg, sp, wcat, bcat)


def _final_body(hp_ref, agg_ref, sp_ref, w_ref, b_ref, o_ref):
    h = jnp.maximum(agg_ref[...] + sp_ref[...], 0.0) + hp_ref[...]
    o_ref[...] = jnp.dot(h, w_ref[...], preferred_element_type=jnp.float32) \
        + b_ref[...][0]


def _tc_final(h_prev, agg, sp, wout, bout):
    blk = 512
    grid = (NPAD // blk,)
    return pl.pallas_call(
        _final_body,
        grid=grid,
        in_specs=[
            pl.BlockSpec((blk, 128), lambda i: (i, 0)),
            pl.BlockSpec((blk, 128), lambda i: (i, 0)),
            pl.BlockSpec((blk, 128), lambda i: (i, 0)),
            pl.BlockSpec((128, OUT), lambda i: (0, 0)),
            pl.BlockSpec((8, OUT), lambda i: (0, 0)),
        ],
        out_specs=pl.BlockSpec((blk, OUT), lambda i: (i, 0)),
        out_shape=jax.ShapeDtypeStruct((NPAD, OUT), jnp.float32),
    )(h_prev, agg, sp, wout, bout)


def _sc_body(q_hbm, kv_hbm, srcs_hbm, offs_hbm, out_hbm,
             offs_v, q_v, out_v, idx_v, kv_v, st_m, st_s, st_a, sem):
    wid = lax.axis_index("s") * 2 + lax.axis_index("c")
    nstart = wid * NODES_PER_W

    pltpu.sync_copy(offs_hbm.at[pl.ds(nstart, NODES_PER_W + 16)], offs_v)
    pltpu.sync_copy(q_hbm.at[pl.ds(nstart * 128, NODES_PER_W * 128)], q_v)

    lanes = lax.iota(jnp.int32, 16)

    def node_body(i, _):
        oo = offs_v[pl.ds(i, 16)]
        s = oo[0]
        e = oo[1]
        base = (s // 16) * 16
        ntiles = jnp.where(e > s, (e - base + 15) // 16, 0)

        def init_body(h, _):
            h16 = h * 16
            st_m[pl.ds(h16, 16)] = jnp.full((16,), NEG, jnp.float32)
            st_s[pl.ds(h16, 16)] = jnp.zeros((16,), jnp.float32)
            st_a[pl.ds(h16, 16)] = jnp.zeros((16,), jnp.float32)
            return 0

        lax.fori_loop(0, HEADS, init_body, 0)

        refetch = ntiles > CACHE_T
        pltpu.sync_copy(srcs_hbm.at[pl.ds(base, CACHE_T * 16)], idx_v)

        def tile_alpha(slot, h16, valid):
            qh = q_v[pl.ds(i * 128 + h16, 16)]
            rows = slot * 16 + lanes
            ps = []
            for c in range(HC):
                col = plsc.load_gather(
                    kv_v, [rows, jnp.full((16,), h16 + c, jnp.int32)])
                ps.append(qh[c] * col)
            while len(ps) > 1:
                ps = [ps[k] + ps[k + 1] for k in range(0, len(ps), 2)]
            return jnp.where(valid, ps[0], NEG)

        def fetch_tile(t, slot):
            def stage_idx():
                g0 = pl.multiple_of(base + t * 16, 16)
                pltpu.sync_copy(srcs_hbm.at[pl.ds(g0, 16)],
                                idx_v.at[pl.ds(slot * 16, 16)])

            pl.when(refetch)(stage_idx)
            idxreg = idx_v[pl.ds(slot * 16, 16)]
            pltpu.async_copy(
                kv_hbm.at[idxreg], kv_v.at[pl.ds(slot * 16, 16)], sem).wait()

        def tile_valid(t):
            gvec = base + t * 16 + lanes
            return (gvec >= s) & (gvec < e)

        def max_body(t, _):
            slot = lax.rem(t, CACHE_T)
            fetch_tile(t, slot)
            valid = tile_valid(t)

            def head_body(h, _):
                h16 = h * 16
                alpha = tile_alpha(slot, h16, valid)
                st_m[pl.ds(h16, 16)] = jnp.maximum(st_m[pl.ds(h16, 16)], alpha)
                return 0

            lax.fori_loop(0, HEADS, head_body, 0)
            return 0

        lax.fori_loop(0, ntiles, max_body, 0)

        def red_body(h, _):
            h16 = h * 16
            st_m[pl.ds(h16, 16)] = jnp.full(
                (16,), jnp.max(st_m[pl.ds(h16, 16)]), jnp.float32)
            return 0

        lax.fori_loop(0, HEADS, red_body, 0)

        def den_body(t, _):
            slot = lax.rem(t, CACHE_T)
            pl.when(refetch)(lambda: fetch_tile(t, slot))
            valid = tile_valid(t)

            def head_body(h, _):
                h16 = h * 16
                alpha = tile_alpha(slot, h16, valid)
                w = jnp.where(valid, jnp.exp(alpha - st_m[pl.ds(h16, 16)]), 0.0)
                d = st_s[pl.ds(h16, 16)][0]
                for j in range(16):
                    d = d + w[j]
                st_s[pl.ds(h16, 16)] = jnp.full((16,), d, jnp.float32)
                return 0

            lax.fori_loop(0, HEADS, head_body, 0)
            return 0

        lax.fori_loop(0, ntiles, den_body, 0)

        def acc_body(t, _):
            slot = lax.rem(t, CACHE_T)
            pl.when(refetch)(lambda: fetch_tile(t, slot))
            valid = tile_valid(t)

            def head_body(h, _):
                h16 = h * 16
                alpha = tile_alpha(slot, h16, valid)
                w = jnp.where(valid, jnp.exp(alpha - st_m[pl.ds(h16, 16)]), 0.0)
                attn = w / (st_s[pl.ds(h16, 16)] + 1e-16)
                acc = st_a[pl.ds(h16, 16)]
                vcols = 128 + h16 + lanes
                for j in range(16):
                    vrow = plsc.load_gather(
                        kv_v, [slot * 16 + j + jnp.zeros((16,), jnp.int32),
                               vcols])
                    acc = acc + attn[j] * vrow
                st_a[pl.ds(h16, 16)] = acc
                return 0

            lax.fori_loop(0, HEADS, head_body, 0)
            return 0

        lax.fori_loop(0, ntiles, acc_body, 0)

        def fin_body(h, _):
            h16 = h * 16
            out_v[pl.ds(i * 128 + h16, 16)] = st_a[pl.ds(h16, 16)]
            return 0

        lax.fori_loop(0, HEADS, fin_body, 0)
        return 0

    lax.fori_loop(0, NODES_PER_W, node_body, 0)
    pltpu.sync_copy(out_v, out_hbm.at[pl.ds(nstart * 128, NODES_PER_W * 128)])


@functools.partial(
    pl.kernel,
    out_type=jax.ShapeDtypeStruct((NPAD * 128,), jnp.float32),
    mesh=plsc.VectorSubcoreMesh(core_axis_name="c", subcore_axis_name="s"),
    compiler_params=pltpu.CompilerParams(needs_layout_passes=False),
    scratch_types=[
        pltpu.VMEM((NODES_PER_W + 16,), jnp.int32),
        pltpu.VMEM((NODES_PER_W * 128,), jnp.float32),
        pltpu.VMEM((NODES_PER_W * 128,), jnp.float32),
        pltpu.VMEM((CACHE_T * 16,), jnp.int32),
        pltpu.VMEM((CACHE_T * 16, 256), jnp.float32),
        pltpu.VMEM((128,), jnp.float32),
        pltpu.VMEM((128,), jnp.float32),
        pltpu.VMEM((128,), jnp.float32),
        pltpu.SemaphoreType.DMA,
    ],
)
def _sc_agg(q_hbm, kv_hbm, srcs_hbm, offs_hbm, out_hbm,
            offs_v, q_v, out_v, idx_v, kv_v, st_m, st_s, st_a, sem):
    _sc_body(q_hbm, kv_hbm, srcs_hbm, offs_hbm, out_hbm,
             offs_v, q_v, out_v, idx_v, kv_v, st_m, st_s, st_a, sem)


def kernel(x, edge_index, batch, Wq, bq, Wk, bk, Wv, bv, Ws, bs, Wout, bout):
    src = edge_index[0]
    dst = edge_index[1]
    perm = jnp.argsort(dst)
    ssrc = jnp.pad(src[perm].astype(jnp.int32), (0, CACHE_T * 16))
    sdst = dst[perm]
    offs = jnp.searchsorted(
        sdst, jnp.arange(OFFS_PAD, dtype=jnp.int32), side="left"
    ).astype(jnp.int32)

    x_pad = jnp.pad(x, ((0, NPAD - N), (0, 0)))
    wcat = jnp.concatenate([Wq * 0.25, Wk, Wv, Ws], axis=2)
    bcat = jnp.broadcast_to(
        jnp.concatenate([bq * 0.25, bk, bv, bs], axis=1)[:, None, :],
        (L, 8, 512))

    def layer(carry, lw):
        h, agg, sp = carry
        wc, bc = lw
        h, q, kv, sp = _tc_proj(h, agg, sp, wc, bc)
        agg = _sc_agg(q.reshape(-1), kv, ssrc, offs).reshape(NPAD, 128)
        return (h, agg, sp), 0.0

    init = (x_pad, jnp.zeros((NPAD, 128), jnp.float32),
            jnp.zeros((NPAD, 128), jnp.float32))
    (h, agg, sp), _ = lax.scan(layer, init, (wcat, bcat))
    out = _tc_final(h, agg, sp, Wout, jnp.broadcast_to(bout, (8, OUT)))
    return out[:N]
